# baseline (speedup 1.0000x reference)
import functools

import jax
import jax.numpy as jnp
from jax import lax
from jax.experimental import pallas as pl
from jax.experimental.pallas import tpu as pltpu
from jax.experimental.pallas import tpu_sc as plsc

NN = 50000
EE = 800000
FF = 64
GG = 256
NLAYER = 3

RB = 400
NRB = NN // RB
EB = 1000
NEB = EE // 2 // EB

NCORES = 2
NSUB = 16
NW = NCORES * NSUB
CH = 64
EPW = 25088
NCHU = EPW // CH
LASTCH = (EE - (NW - 1) * EPW) // CH
NP = 50048
RPT = NP // NSUB
NDF = RPT // CH
DTL = RPT - NDF * CH

_C0 = 6.937419924957222e-05
_C1 = 0.9962627288195075
_C2 = -0.4664446246706749
_C3 = 0.2186675662990867
_C4 = -0.05545986954189676


def _pad8(v):
    return jnp.zeros((8, v.shape[-1]), jnp.float32).at[0].set(v)


def _ilv(a, b):
    return jnp.stack([a, b], axis=2).reshape(a.shape[0], -1)


def _embed_stats_body(x_ref, w_ref, b_ref, h_ref, st_ref):
    i = pl.program_id(0)
    h = jnp.dot(x_ref[...], w_ref[...], preferred_element_type=jnp.float32)
    h = h + b_ref[0:1, :]
    h_ref[...] = h
    s0 = jnp.sum(h, axis=0)[None, :]
    s1 = jnp.sum(h * h, axis=0)[None, :]
    st = jnp.concatenate([s0, s1, jnp.zeros((6, FF), jnp.float32)], axis=0)

    @pl.when(i == 0)
    def _():
        st_ref[...] = st

    @pl.when(i != 0)
    def _():
        st_ref[...] += st


_embed_call = pl.pallas_call(
    _embed_stats_body,
    grid=(NRB,),
    in_specs=[
        pl.BlockSpec((RB, 128), lambda i: (i, 0)),
        pl.BlockSpec((128, FF), lambda i: (0, 0)),
        pl.BlockSpec((8, FF), lambda i: (0, 0)),
    ],
    out_specs=[
        pl.BlockSpec((RB, FF), lambda i: (i, 0)),
        pl.BlockSpec((8, FF), lambda i: (0, 0)),
    ],
    out_shape=[
        jax.ShapeDtypeStruct((NN, FF), jnp.float32),
        jax.ShapeDtypeStruct((8, FF), jnp.float32),
    ],
)


def _bnproj_body(h_ref, st_ref, bnp_ref, w_ref,
                 hbn_ref, p0s_ref, p0d_ref, p1s_ref, p1d_ref):
    mu = st_ref[0:1, :] / NN
    var = st_ref[1:2, :] / NN - mu * mu
    inv = lax.rsqrt(var + 1e-5)
    hbn = (h_ref[...] - mu) * (inv * bnp_ref[0:1, :]) + bnp_ref[1:2, :]
    hbn_ref[...] = hbn
    w = w_ref[...]
    for t, ref in enumerate((p0s_ref, p0d_ref, p1s_ref, p1d_ref)):
        ref[...] = jnp.dot(hbn, w[:, 64 * t:64 * (t + 1)],
                           preferred_element_type=jnp.float32)


_bnproj_call = pl.pallas_call(
    _bnproj_body,
    grid=(NRB,),
    in_specs=[
        pl.BlockSpec((RB, FF), lambda i: (i, 0)),
        pl.BlockSpec((8, FF), lambda i: (0, 0)),
        pl.BlockSpec((8, FF), lambda i: (0, 0)),
        pl.BlockSpec((FF, 256), lambda i: (0, 0)),
    ],
    out_specs=[pl.BlockSpec((RB, FF), lambda i: (i, 0))] * 5,
    out_shape=[jax.ShapeDtypeStruct((NN, FF), jnp.float32)] * 5,
)


def _ea_body(ea_ref, w_ref, b_ref, *out_refs):
    z = jnp.dot(ea_ref[...], w_ref[...], preferred_element_type=jnp.float32)
    z = z + b_ref[0:1, :]
    for t in range(6):
        out_refs[t][...] = jnp.concatenate(
            [z[:, 64 * t:64 * (t + 1)], z[:, 384 + 64 * t:384 + 64 * (t + 1)]],
            axis=1)


_ea_call = pl.pallas_call(
    _ea_body,
    grid=(NEB,),
    in_specs=[
        pl.BlockSpec((EB, 32), lambda i: (i, 0)),
        pl.BlockSpec((32, 768), lambda i: (0, 0)),
        pl.BlockSpec((8, 768), lambda i: (0, 0)),
    ],
    out_specs=[pl.BlockSpec((EB, 128), lambda i: (i, 0))] * 6,
    out_shape=[jax.ShapeDtypeStruct((EE // 2, 128), jnp.float32)] * 6,
)


def _combine_body(a_ref, b_ref, c_ref, d_ref, agg_ref, st_ref):
    i = pl.program_id(0)
    agg = jnp.concatenate([a_ref[0, 0] + b_ref[0, 0], c_ref[0, 0] + d_ref[0, 0]],
                          axis=1)
    agg_ref[...] = agg
    s0 = jnp.sum(agg, axis=0)[None, :]
    s1 = jnp.sum(agg * agg, axis=0)[None, :]
    st = jnp.concatenate([s0, s1, jnp.zeros((6, FF), jnp.float32)], axis=0)

    @pl.when(i == 0)
    def _():
        st_ref[...] = st

    @pl.when(i != 0)
    def _():
        st_ref[...] += st


_combine_call = pl.pallas_call(
    _combine_body,
    grid=(NRB,),
    in_specs=[
        pl.BlockSpec((1, 1, RB, 32), lambda i: (0, 0, i, 0)),
        pl.BlockSpec((1, 1, RB, 32), lambda i: (1, 0, i, 0)),
        pl.BlockSpec((1, 1, RB, 32), lambda i: (0, 1, i, 0)),
        pl.BlockSpec((1, 1, RB, 32), lambda i: (1, 1, i, 0)),
    ],
    out_specs=[
        pl.BlockSpec((RB, FF), lambda i: (i, 0)),
        pl.BlockSpec((8, FF), lambda i: (0, 0)),
    ],
    out_shape=[
        jax.ShapeDtypeStruct((NN, FF), jnp.float32),
        jax.ShapeDtypeStruct((8, FF), jnp.float32),
    ],
)


def _post_body(agg_ref, st_ref, bnp_ref, hbn_ref, h_ref, stn_ref):
    i = pl.program_id(0)
    mu = st_ref[0:1, :] / NN
    var = st_ref[1:2, :] / NN - mu * mu
    inv = lax.rsqrt(var + 1e-5)
    o = (agg_ref[...] - mu) * (inv * bnp_ref[0:1, :]) + bnp_ref[1:2, :]
    h = jax.nn.softplus(o + hbn_ref[...])
    h_ref[...] = h
    s0 = jnp.sum(h, axis=0)[None, :]
    s1 = jnp.sum(h * h, axis=0)[None, :]
    st = jnp.concatenate([s0, s1, jnp.zeros((6, FF), jnp.float32)], axis=0)

    @pl.when(i == 0)
    def _():
        stn_ref[...] = st

    @pl.when(i != 0)
    def _():
        stn_ref[...] += st


_post_call = pl.pallas_call(
    _post_body,
    grid=(NRB,),
    in_specs=[
        pl.BlockSpec((RB, FF), lambda i: (i, 0)),
        pl.BlockSpec((8, FF), lambda i: (0, 0)),
        pl.BlockSpec((8, FF), lambda i: (0, 0)),
        pl.BlockSpec((RB, FF), lambda i: (i, 0)),
    ],
    out_specs=[
        pl.BlockSpec((RB, FF), lambda i: (i, 0)),
        pl.BlockSpec((8, FF), lambda i: (0, 0)),
    ],
    out_shape=[
        jax.ShapeDtypeStruct((NN, FF), jnp.float32),
        jax.ShapeDtypeStruct((8, FF), jnp.float32),
    ],
)


def _pool_body(h_ref, b_ref, pooled_ref, cnt_ref):
    i = pl.program_id(0)
    bid = b_ref[0, 0, :]
    oh = (bid[:, None] == lax.broadcasted_iota(jnp.int32, (RB, GG), 1))
    oh = oh.astype(jnp.float32)
    ps = lax.dot_general(oh, h_ref[...], (((0,), (0,)), ((), ())),
                         preferred_element_type=jnp.float32)
    cnt = jnp.sum(oh, axis=0)[None, :]
    cnt = jnp.concatenate([cnt, jnp.zeros((7, GG), jnp.float32)], axis=0)

    @pl.when(i == 0)
    def _():
        pooled_ref[...] = ps
        cnt_ref[...] = cnt

    @pl.when(i != 0)
    def _():
        pooled_ref[...] += ps
        cnt_ref[...] += cnt


_pool_call = pl.pallas_call(
    _pool_body,
    grid=(NRB,),
    in_specs=[
        pl.BlockSpec((RB, FF), lambda i: (i, 0)),
        pl.BlockSpec((1, 1, RB), lambda i: (i, 0, 0)),
    ],
    out_specs=[
        pl.BlockSpec((GG, FF), lambda i: (0, 0)),
        pl.BlockSpec((8, GG), lambda i: (0, 0)),
    ],
    out_shape=[
        jax.ShapeDtypeStruct((GG, FF), jnp.float32),
        jax.ShapeDtypeStruct((8, GG), jnp.float32),
    ],
)


def _head_body(pooled_ref, cnt_ref, w1_ref, b1_ref, w2_ref, b2_ref, out_ref):
    cnt = jnp.maximum(cnt_ref[0:1, :], 1.0)
    crys = pooled_ref[...] / cnt.reshape(GG, 1)
    crys = jax.nn.softplus(crys)
    crys = jnp.dot(crys, w1_ref[...], preferred_element_type=jnp.float32)
    crys = jax.nn.softplus(crys + b1_ref[0:1, :])
    out = jnp.dot(crys, w2_ref[...], preferred_element_type=jnp.float32)
    out_ref[...] = out + b2_ref[0, 0]


_head_call = pl.pallas_call(
    _head_body,
    grid=(1,),
    in_specs=[
        pl.BlockSpec((GG, FF), lambda i: (0, 0)),
        pl.BlockSpec((8, GG), lambda i: (0, 0)),
        pl.BlockSpec((FF, 128), lambda i: (0, 0)),
        pl.BlockSpec((8, 128), lambda i: (0, 0)),
        pl.BlockSpec((128, 1), lambda i: (0, 0)),
        pl.BlockSpec((8, 8), lambda i: (0, 0)),
    ],
    out_specs=pl.BlockSpec((GG, 1), lambda i: (0, 0)),
    out_shape=jax.ShapeDtypeStruct((GG, 1), jnp.float32),
)


_mesh = plsc.VectorSubcoreMesh(core_axis_name="c", subcore_axis_name="s")


def _edge_sc_body(src_hbm, dst_hbm, zeros_hbm, ps0, pd0, ea0, ps1, pd1, ea1,
                  out_hbm,
                  ixs0, ixd0, gs0, gd0, ea0v, ixs1, ixd1, gs1, gd1, ea1v, mv,
                  acc, sixs0, sixd0, sgs0, sgd0, sea0,
                  sixs1, sixd1, sgs1, sgd1, sea1):
    c = lax.axis_index("c")
    s = lax.axis_index("s")
    wid = c * NSUB + s
    base = wid * EPW
    nch2 = jnp.where(wid == NW - 1, LASTCH // 2, NCHU // 2)

    slot = (
        (ixs0, ixd0, gs0, gd0, ea0v, sixs0, sixd0, sgs0, sgd0, sea0),
        (ixs1, ixd1, gs1, gd1, ea1v, sixs1, sixd1, sgs1, sgd1, sea1),
    )

    def issue_idx(e0, t):
        ixs, ixd = slot[t][0], slot[t][1]
        pltpu.async_copy(src_hbm.at[pl.ds(e0, CH)], ixs, slot[t][5])
        pltpu.async_copy(dst_hbm.at[pl.ds(e0, CH)], ixd, slot[t][6])

    def wait_idx(e0, t):
        pltpu.make_async_copy(src_hbm.at[pl.ds(e0, CH)], slot[t][0],
                              slot[t][5]).wait()
        pltpu.make_async_copy(dst_hbm.at[pl.ds(e0, CH)], slot[t][1],
                              slot[t][6]).wait()

    def issue_gather(e0, t, ps, pd, eah):
        ixs, ixd, gsv, gdv, eav = slot[t][:5]
        pltpu.async_copy(ps.at[ixs], gsv, slot[t][7])
        pltpu.async_copy(pd.at[ixd], gdv, slot[t][8])
        pltpu.async_copy(eah.at[pl.ds(e0 // 2, CH // 2)], eav, slot[t][9])

    def wait_gather(e0, t, ps, pd, eah):
        ixs, ixd, gsv, gdv, eav = slot[t][:5]
        pltpu.make_async_copy(ps.at[ixs], gsv, slot[t][7]).wait()
        pltpu.make_async_copy(pd.at[ixd], gdv, slot[t][8]).wait()
        pltpu.make_async_copy(eah.at[pl.ds(e0 // 2, CH // 2)], eav,
                              slot[t][9]).wait()

    def compute_scatter(t):
        ixs, ixd, gsv, gdv, eav = slot[t][:5]

        @plsc.parallel_loop(0, CH, 1, unroll=16)
        def _row(r):
            r2 = r // 2
            cb = (r % 2) * 64
            for q in (0, 1):
                af = (gsv[r, pl.ds(q * 16, 16)] + gdv[r, pl.ds(q * 16, 16)]
                      + eav[r2, pl.ds(cb + q * 16, 16)])
                asv = (gsv[r, pl.ds(32 + q * 16, 16)]
                       + gdv[r, pl.ds(32 + q * 16, 16)]
                       + eav[r2, pl.ds(cb + 32 + q * 16, 16)])
                sig = 1.0 / (1.0 + jnp.exp(-af))
                t2 = jnp.exp(-jnp.abs(asv))
                poly = _C0 + t2 * (_C1 + t2 * (_C2 + t2 * (_C3 + t2 * _C4)))
                sp = jnp.maximum(asv, 0.0) + poly
                mv[r, pl.ds(q * 16, 16)] = sig * sp

        pltpu.sync_copy(mv, acc.at[ixs], add=True)

    for p, (ps, pd, eah) in enumerate(((ps0, pd0, ea0), (ps1, pd1, ea1))):
        pltpu.sync_copy(zeros_hbm, acc.at[pl.ds(s * RPT, RPT)])
        plsc.subcore_barrier()

        pltpu.sync_copy(src_hbm.at[pl.ds(base, CH)], ixs0)
        pltpu.sync_copy(dst_hbm.at[pl.ds(base, CH)], ixd0)
        issue_gather(base, 0, ps, pd, eah)
        issue_idx(base + CH, 1)

        def step(k, carry):
            ea_ = base + 2 * k * CH
            eb_ = ea_ + CH
            more = k < nch2 - 1
            wait_gather(ea_, 0, ps, pd, eah)
            wait_idx(eb_, 1)
            issue_gather(eb_, 1, ps, pd, eah)
            compute_scatter(0)

            @pl.when(more)
            def _():
                issue_idx(eb_ + CH, 0)
            wait_gather(eb_, 1, ps, pd, eah)

            @pl.when(more)
            def _():
                wait_idx(eb_ + CH, 0)
                issue_gather(eb_ + CH, 0, ps, pd, eah)
            compute_scatter(1)

            @pl.when(more)
            def _():
                issue_idx(eb_ + 2 * CH, 1)
            return carry

        lax.fori_loop(0, nch2, step, 0)
        plsc.subcore_barrier()

        def dump(j, carry):
            r0 = s * RPT + j * CH
            pltpu.sync_copy(acc.at[pl.ds(r0, CH)], mv)
            pltpu.sync_copy(mv, out_hbm.at[c, p, pl.ds(r0, CH)])
            return carry

        lax.fori_loop(0, NDF, dump, 0)
        r0 = s * RPT + NDF * CH
        pltpu.sync_copy(acc.at[pl.ds(r0, DTL)], mv.at[pl.ds(0, DTL)])
        pltpu.sync_copy(mv.at[pl.ds(0, DTL)], out_hbm.at[c, p, pl.ds(r0, DTL)])
        plsc.subcore_barrier()


_edge_call = functools.partial(
    pl.kernel,
    mesh=_mesh,
    compiler_params=pltpu.CompilerParams(use_tc_tiling_on_sc=False),
    out_type=jax.ShapeDtypeStruct((NCORES, 2, NP, 32), jnp.float32),
    scratch_types=(
        [
            pltpu.VMEM((CH,), jnp.int32),
            pltpu.VMEM((CH,), jnp.int32),
            pltpu.VMEM((CH, FF), jnp.float32),
            pltpu.VMEM((CH, FF), jnp.float32),
            pltpu.VMEM((CH // 2, 128), jnp.float32),
        ] * 2
        + [
            pltpu.VMEM((CH, 32), jnp.float32),
            pltpu.VMEM_SHARED((NP, 32), jnp.float32),
        ]
        + [pltpu.SemaphoreType.DMA] * 10
    ),
)(_edge_sc_body)


def _edge_stage(src, dst, zeros_rpt, ps0, pd0, ea0, ps1, pd1, ea1):
    return _edge_call(src, dst, zeros_rpt, ps0, pd0, ea0, ps1, pd1, ea1)


def kernel(x, edge_index, edge_attr, batch, emb_W, emb_b, lin_f_W, lin_f_b,
           lin_s_W, lin_s_b, bn1_g, bn1_b, bn2_g, bn2_b, fc1_W, fc1_b,
           out_W, out_b):
    src = edge_index[0]
    dst = edge_index[1]
    zeros_rpt = jnp.zeros((RPT, 32), jnp.float32)

    wea_cols = []
    bea_cols = []
    for l in range(NLAYER):
        w3f = lin_f_W[l][128:144]
        w3s = lin_s_W[l][128:144]
        for p in (0, 1):
            cp = slice(32 * p, 32 * p + 32)
            wea_cols.append(jnp.concatenate([w3f[:, cp], w3s[:, cp]], axis=1))
            bea_cols.append(jnp.concatenate([lin_f_b[l][cp], lin_s_b[l][cp]]))
    wea = jnp.concatenate(wea_cols, axis=1)
    bea1 = jnp.concatenate(bea_cols)
    zb = jnp.zeros_like(wea)
    wea2 = jnp.concatenate(
        [jnp.concatenate([wea, zb], axis=1), jnp.concatenate([zb, wea], axis=1)],
        axis=0)
    bea2 = _pad8(jnp.concatenate([bea1, bea1]))
    ea6 = _ea_call(edge_attr.reshape(EE // 2, 32), wea2, bea2)

    h, st = _embed_call(x, emb_W, _pad8(emb_b))

    for l in range(NLAYER):
        w1f = lin_f_W[l][0:64]
        w2f = lin_f_W[l][64:128]
        w1s = lin_s_W[l][0:64]
        w2s = lin_s_W[l][64:128]
        cols = []
        for p in (0, 1):
            cp = slice(32 * p, 32 * p + 32)
            cols.append(jnp.concatenate([w1f[:, cp], w1s[:, cp]], axis=1))
            cols.append(jnp.concatenate([w2f[:, cp], w2s[:, cp]], axis=1))
        w256 = jnp.concatenate(cols, axis=1)
        bnp1 = jnp.zeros((8, FF), jnp.float32).at[0].set(bn1_g[l]).at[1].set(bn1_b[l])
        hbn, ps0, pd0, ps1, pd1 = _bnproj_call(h, st, bnp1, w256)

        eout = _edge_stage(src, dst, zeros_rpt, ps0, pd0, ea6[2 * l],
                           ps1, pd1, ea6[2 * l + 1])
        agg, st2 = _combine_call(eout, eout, eout, eout)
        bnp2 = jnp.zeros((8, FF), jnp.float32).at[0].set(bn2_g[l]).at[1].set(bn2_b[l])
        h, st = _post_call(agg, st2, bnp2, hbn)

    batch3d = batch.reshape(NRB, 1, RB)
    pooled, counts = _pool_call(h, batch3d)
    out = _head_call(pooled, counts, fc1_W, _pad8(fc1_b), out_W,
                     jnp.zeros((8, 8), jnp.float32).at[0, 0].set(out_b[0]))
    return out


# speedup vs baseline: 1.9489x; 1.9489x over previous
import functools

import jax
import jax.numpy as jnp
from jax import lax
from jax.experimental import pallas as pl
from jax.experimental.pallas import tpu as pltpu
from jax.experimental.pallas import tpu_sc as plsc

NN = 50000
EE = 800000
FF = 64
GG = 256
NLAYER = 3

RB = 400
NRB = NN // RB
EB = 1000
NEB = EE // 2 // EB

NCORES = 2
NSUB = 16
NW = NCORES * NSUB
CH = 64
EPW = 25088
NCHU = EPW // CH
LASTCH = (EE - (NW - 1) * EPW) // CH
NP = 50048
RPT = NP // NSUB
NDF = RPT // CH
DTL = RPT - NDF * CH

_C0 = 6.937419924957222e-05
_C1 = 0.9962627288195075
_C2 = -0.4664446246706749
_C3 = 0.2186675662990867
_C4 = -0.05545986954189676


def _pad8(v):
    return jnp.zeros((8, v.shape[-1]), jnp.float32).at[0].set(v)


def _ilv(a, b):
    return jnp.stack([a, b], axis=2).reshape(a.shape[0], -1)


def _embed_stats_body(x_ref, w_ref, b_ref, h_ref, st_ref):
    i = pl.program_id(0)
    h = jnp.dot(x_ref[...], w_ref[...], preferred_element_type=jnp.float32)
    h = h + b_ref[0:1, :]
    h_ref[...] = h
    s0 = jnp.sum(h, axis=0)[None, :]
    s1 = jnp.sum(h * h, axis=0)[None, :]
    st = jnp.concatenate([s0, s1, jnp.zeros((6, FF), jnp.float32)], axis=0)

    @pl.when(i == 0)
    def _():
        st_ref[...] = st

    @pl.when(i != 0)
    def _():
        st_ref[...] += st


_embed_call = pl.pallas_call(
    _embed_stats_body,
    grid=(NRB,),
    in_specs=[
        pl.BlockSpec((RB, 128), lambda i: (i, 0)),
        pl.BlockSpec((128, FF), lambda i: (0, 0)),
        pl.BlockSpec((8, FF), lambda i: (0, 0)),
    ],
    out_specs=[
        pl.BlockSpec((RB, FF), lambda i: (i, 0)),
        pl.BlockSpec((8, FF), lambda i: (0, 0)),
    ],
    out_shape=[
        jax.ShapeDtypeStruct((NN, FF), jnp.float32),
        jax.ShapeDtypeStruct((8, FF), jnp.float32),
    ],
)


def _bnproj_body(h_ref, st_ref, bnp_ref, w_ref,
                 hbn_ref, p0s_ref, p0d_ref, p1s_ref, p1d_ref):
    mu = st_ref[0:1, :] / NN
    var = st_ref[1:2, :] / NN - mu * mu
    inv = lax.rsqrt(var + 1e-5)
    hbn = (h_ref[...] - mu) * (inv * bnp_ref[0:1, :]) + bnp_ref[1:2, :]
    hbn_ref[...] = hbn
    w = w_ref[...]
    for t, ref in enumerate((p0s_ref, p0d_ref, p1s_ref, p1d_ref)):
        ref[...] = jnp.dot(hbn, w[:, 64 * t:64 * (t + 1)],
                           preferred_element_type=jnp.float32)


_bnproj_call = pl.pallas_call(
    _bnproj_body,
    grid=(NRB,),
    in_specs=[
        pl.BlockSpec((RB, FF), lambda i: (i, 0)),
        pl.BlockSpec((8, FF), lambda i: (0, 0)),
        pl.BlockSpec((8, FF), lambda i: (0, 0)),
        pl.BlockSpec((FF, 256), lambda i: (0, 0)),
    ],
    out_specs=[pl.BlockSpec((RB, FF), lambda i: (i, 0))] * 5,
    out_shape=[jax.ShapeDtypeStruct((NN, FF), jnp.float32)] * 5,
)


def _ea_body(ea_ref, w_ref, b_ref, *out_refs):
    z = jnp.dot(ea_ref[...], w_ref[...], preferred_element_type=jnp.float32)
    z = z + b_ref[0:1, :]
    for t in range(6):
        out_refs[t][...] = jnp.concatenate(
            [z[:, 64 * t:64 * (t + 1)], z[:, 384 + 64 * t:384 + 64 * (t + 1)]],
            axis=1)


_ea_call = pl.pallas_call(
    _ea_body,
    grid=(NEB,),
    in_specs=[
        pl.BlockSpec((EB, 32), lambda i: (i, 0)),
        pl.BlockSpec((32, 768), lambda i: (0, 0)),
        pl.BlockSpec((8, 768), lambda i: (0, 0)),
    ],
    out_specs=[pl.BlockSpec((EB, 128), lambda i: (i, 0))] * 6,
    out_shape=[jax.ShapeDtypeStruct((EE // 2, 128), jnp.float32)] * 6,
)


def _combine_body(a_ref, b_ref, c_ref, d_ref, agg_ref, st_ref):
    i = pl.program_id(0)
    agg = jnp.concatenate([a_ref[0, 0] + b_ref[0, 0], c_ref[0, 0] + d_ref[0, 0]],
                          axis=1)
    agg_ref[...] = agg
    s0 = jnp.sum(agg, axis=0)[None, :]
    s1 = jnp.sum(agg * agg, axis=0)[None, :]
    st = jnp.concatenate([s0, s1, jnp.zeros((6, FF), jnp.float32)], axis=0)

    @pl.when(i == 0)
    def _():
        st_ref[...] = st

    @pl.when(i != 0)
    def _():
        st_ref[...] += st


_combine_call = pl.pallas_call(
    _combine_body,
    grid=(NRB,),
    in_specs=[
        pl.BlockSpec((1, 1, RB, 32), lambda i: (0, 0, i, 0)),
        pl.BlockSpec((1, 1, RB, 32), lambda i: (1, 0, i, 0)),
        pl.BlockSpec((1, 1, RB, 32), lambda i: (0, 1, i, 0)),
        pl.BlockSpec((1, 1, RB, 32), lambda i: (1, 1, i, 0)),
    ],
    out_specs=[
        pl.BlockSpec((RB, FF), lambda i: (i, 0)),
        pl.BlockSpec((8, FF), lambda i: (0, 0)),
    ],
    out_shape=[
        jax.ShapeDtypeStruct((NN, FF), jnp.float32),
        jax.ShapeDtypeStruct((8, FF), jnp.float32),
    ],
)


def _post_body(agg_ref, st_ref, bnp_ref, hbn_ref, h_ref, stn_ref):
    i = pl.program_id(0)
    mu = st_ref[0:1, :] / NN
    var = st_ref[1:2, :] / NN - mu * mu
    inv = lax.rsqrt(var + 1e-5)
    o = (agg_ref[...] - mu) * (inv * bnp_ref[0:1, :]) + bnp_ref[1:2, :]
    h = jax.nn.softplus(o + hbn_ref[...])
    h_ref[...] = h
    s0 = jnp.sum(h, axis=0)[None, :]
    s1 = jnp.sum(h * h, axis=0)[None, :]
    st = jnp.concatenate([s0, s1, jnp.zeros((6, FF), jnp.float32)], axis=0)

    @pl.when(i == 0)
    def _():
        stn_ref[...] = st

    @pl.when(i != 0)
    def _():
        stn_ref[...] += st


_post_call = pl.pallas_call(
    _post_body,
    grid=(NRB,),
    in_specs=[
        pl.BlockSpec((RB, FF), lambda i: (i, 0)),
        pl.BlockSpec((8, FF), lambda i: (0, 0)),
        pl.BlockSpec((8, FF), lambda i: (0, 0)),
        pl.BlockSpec((RB, FF), lambda i: (i, 0)),
    ],
    out_specs=[
        pl.BlockSpec((RB, FF), lambda i: (i, 0)),
        pl.BlockSpec((8, FF), lambda i: (0, 0)),
    ],
    out_shape=[
        jax.ShapeDtypeStruct((NN, FF), jnp.float32),
        jax.ShapeDtypeStruct((8, FF), jnp.float32),
    ],
)


def _pool_body(h_ref, b_ref, pooled_ref, cnt_ref):
    i = pl.program_id(0)
    bid = b_ref[0, 0, :]
    oh = (bid[:, None] == lax.broadcasted_iota(jnp.int32, (RB, GG), 1))
    oh = oh.astype(jnp.float32)
    ps = lax.dot_general(oh, h_ref[...], (((0,), (0,)), ((), ())),
                         preferred_element_type=jnp.float32)
    cnt = jnp.sum(oh, axis=0)[None, :]
    cnt = jnp.concatenate([cnt, jnp.zeros((7, GG), jnp.float32)], axis=0)

    @pl.when(i == 0)
    def _():
        pooled_ref[...] = ps
        cnt_ref[...] = cnt

    @pl.when(i != 0)
    def _():
        pooled_ref[...] += ps
        cnt_ref[...] += cnt


_pool_call = pl.pallas_call(
    _pool_body,
    grid=(NRB,),
    in_specs=[
        pl.BlockSpec((RB, FF), lambda i: (i, 0)),
        pl.BlockSpec((1, 1, RB), lambda i: (i, 0, 0)),
    ],
    out_specs=[
        pl.BlockSpec((GG, FF), lambda i: (0, 0)),
        pl.BlockSpec((8, GG), lambda i: (0, 0)),
    ],
    out_shape=[
        jax.ShapeDtypeStruct((GG, FF), jnp.float32),
        jax.ShapeDtypeStruct((8, GG), jnp.float32),
    ],
)


def _head_body(pooled_ref, cnt_ref, w1_ref, b1_ref, w2_ref, b2_ref, out_ref):
    cnt = jnp.maximum(cnt_ref[0:1, :], 1.0)
    crys = pooled_ref[...] / cnt.reshape(GG, 1)
    crys = jax.nn.softplus(crys)
    crys = jnp.dot(crys, w1_ref[...], preferred_element_type=jnp.float32)
    crys = jax.nn.softplus(crys + b1_ref[0:1, :])
    out = jnp.dot(crys, w2_ref[...], preferred_element_type=jnp.float32)
    out_ref[...] = out + b2_ref[0, 0]


_head_call = pl.pallas_call(
    _head_body,
    grid=(1,),
    in_specs=[
        pl.BlockSpec((GG, FF), lambda i: (0, 0)),
        pl.BlockSpec((8, GG), lambda i: (0, 0)),
        pl.BlockSpec((FF, 128), lambda i: (0, 0)),
        pl.BlockSpec((8, 128), lambda i: (0, 0)),
        pl.BlockSpec((128, 1), lambda i: (0, 0)),
        pl.BlockSpec((8, 8), lambda i: (0, 0)),
    ],
    out_specs=pl.BlockSpec((GG, 1), lambda i: (0, 0)),
    out_shape=jax.ShapeDtypeStruct((GG, 1), jnp.float32),
)


_mesh = plsc.VectorSubcoreMesh(core_axis_name="c", subcore_axis_name="s")


def _edge_sc_body(src_hbm, dst_hbm, zeros_hbm, ps0, pd0, ea0, ps1, pd1, ea1,
                  out_hbm,
                  ixs0, ixd0, gs0, gd0, ea0v, ixs1, ixd1, gs1, gd1, ea1v, mv,
                  acc, sixs0, sixd0, sgs0, sgd0, sea0,
                  sixs1, sixd1, sgs1, sgd1, sea1):
    c = lax.axis_index("c")
    s = lax.axis_index("s")
    wid = c * NSUB + s
    base = wid * EPW
    nch2 = jnp.where(wid == NW - 1, LASTCH // 2, NCHU // 2)

    slot = (
        (ixs0, ixd0, gs0, gd0, ea0v, sixs0, sixd0, sgs0, sgd0, sea0),
        (ixs1, ixd1, gs1, gd1, ea1v, sixs1, sixd1, sgs1, sgd1, sea1),
    )

    def issue_idx(e0, t):
        ixs, ixd = slot[t][0], slot[t][1]
        pltpu.async_copy(src_hbm.at[pl.ds(e0, CH)], ixs, slot[t][5])
        pltpu.async_copy(dst_hbm.at[pl.ds(e0, CH)], ixd, slot[t][6])

    def wait_idx(e0, t):
        pltpu.make_async_copy(src_hbm.at[pl.ds(e0, CH)], slot[t][0],
                              slot[t][5]).wait()
        pltpu.make_async_copy(dst_hbm.at[pl.ds(e0, CH)], slot[t][1],
                              slot[t][6]).wait()

    def issue_gather(e0, t, ps, pd, eah):
        ixs, ixd, gsv, gdv, eav = slot[t][:5]
        pltpu.async_copy(ps.at[ixs], gsv, slot[t][7])
        pltpu.async_copy(pd.at[ixd], gdv, slot[t][8])
        pltpu.async_copy(eah.at[pl.ds(e0 // 2, CH // 2)], eav, slot[t][9])

    def wait_gather(e0, t, ps, pd, eah):
        ixs, ixd, gsv, gdv, eav = slot[t][:5]
        pltpu.make_async_copy(ps.at[ixs], gsv, slot[t][7]).wait()
        pltpu.make_async_copy(pd.at[ixd], gdv, slot[t][8]).wait()
        pltpu.make_async_copy(eah.at[pl.ds(e0 // 2, CH // 2)], eav,
                              slot[t][9]).wait()

    def compute_scatter(t):
        ixs, ixd, gsv, gdv, eav = slot[t][:5]

        @plsc.parallel_loop(0, CH, 1, unroll=8)
        def _row(r):
            r2 = r // 2
            cb = (r % 2) * 64
            for q in (0, 1):
                af = (gsv[r, pl.ds(q * 16, 16)] + gdv[r, pl.ds(q * 16, 16)]
                      + eav[r2, pl.ds(cb + q * 16, 16)])
                asv = (gsv[r, pl.ds(32 + q * 16, 16)]
                       + gdv[r, pl.ds(32 + q * 16, 16)]
                       + eav[r2, pl.ds(cb + 32 + q * 16, 16)])
                sig = 1.0 / (1.0 + jnp.exp(-af))
                t2 = jnp.exp(-jnp.abs(asv))
                poly = _C0 + t2 * (_C1 + t2 * (_C2 + t2 * (_C3 + t2 * _C4)))
                sp = jnp.maximum(asv, 0.0) + poly
                mv[r, pl.ds(q * 16, 16)] = sig * sp

        pltpu.sync_copy(mv, acc.at[ixs], add=True)

    for p, (ps, pd, eah) in enumerate(((ps0, pd0, ea0), (ps1, pd1, ea1))):
        pltpu.sync_copy(zeros_hbm, acc.at[pl.ds(s * RPT, RPT)])
        plsc.subcore_barrier()

        pltpu.sync_copy(src_hbm.at[pl.ds(base, CH)], ixs0)
        pltpu.sync_copy(dst_hbm.at[pl.ds(base, CH)], ixd0)
        issue_gather(base, 0, ps, pd, eah)
        issue_idx(base + CH, 1)

        def step(k, carry):
            ea_ = base + 2 * k * CH
            eb_ = ea_ + CH
            more = k < nch2 - 1
            wait_gather(ea_, 0, ps, pd, eah)
            wait_idx(eb_, 1)
            issue_gather(eb_, 1, ps, pd, eah)
            compute_scatter(0)

            @pl.when(more)
            def _():
                issue_idx(eb_ + CH, 0)
            wait_gather(eb_, 1, ps, pd, eah)

            @pl.when(more)
            def _():
                wait_idx(eb_ + CH, 0)
                issue_gather(eb_ + CH, 0, ps, pd, eah)
            compute_scatter(1)

            @pl.when(more)
            def _():
                issue_idx(eb_ + 2 * CH, 1)
            return carry

        lax.fori_loop(0, nch2, step, 0)
        plsc.subcore_barrier()

        def dump(j, carry):
            r0 = s * RPT + j * CH
            pltpu.sync_copy(acc.at[pl.ds(r0, CH)], mv)
            pltpu.sync_copy(mv, out_hbm.at[c, p, pl.ds(r0, CH)])
            return carry

        lax.fori_loop(0, NDF, dump, 0)
        r0 = s * RPT + NDF * CH
        pltpu.sync_copy(acc.at[pl.ds(r0, DTL)], mv.at[pl.ds(0, DTL)])
        pltpu.sync_copy(mv.at[pl.ds(0, DTL)], out_hbm.at[c, p, pl.ds(r0, DTL)])
        plsc.subcore_barrier()


_edge_call = functools.partial(
    pl.kernel,
    mesh=_mesh,
    compiler_params=pltpu.CompilerParams(use_tc_tiling_on_sc=False),
    out_type=jax.ShapeDtypeStruct((NCORES, 2, NP, 32), jnp.float32),
    scratch_types=(
        [
            pltpu.VMEM((CH,), jnp.int32),
            pltpu.VMEM((CH,), jnp.int32),
            pltpu.VMEM((CH, FF), jnp.float32),
            pltpu.VMEM((CH, FF), jnp.float32),
            pltpu.VMEM((CH // 2, 128), jnp.float32),
        ] * 2
        + [
            pltpu.VMEM((CH, 32), jnp.float32),
            pltpu.VMEM_SHARED((NP, 32), jnp.float32),
        ]
        + [pltpu.SemaphoreType.DMA] * 10
    ),
)(_edge_sc_body)


def _edge_stage(src, dst, zeros_rpt, ps0, pd0, ea0, ps1, pd1, ea1):
    return _edge_call(src, dst, zeros_rpt, ps0, pd0, ea0, ps1, pd1, ea1)


def kernel(x, edge_index, edge_attr, batch, emb_W, emb_b, lin_f_W, lin_f_b,
           lin_s_W, lin_s_b, bn1_g, bn1_b, bn2_g, bn2_b, fc1_W, fc1_b,
           out_W, out_b):
    src = edge_index[0]
    dst = edge_index[1]
    zeros_rpt = jnp.zeros((RPT, 32), jnp.float32)

    wea_cols = []
    bea_cols = []
    for l in range(NLAYER):
        w3f = lin_f_W[l][128:144]
        w3s = lin_s_W[l][128:144]
        for p in (0, 1):
            cp = slice(32 * p, 32 * p + 32)
            wea_cols.append(jnp.concatenate([w3f[:, cp], w3s[:, cp]], axis=1))
            bea_cols.append(jnp.concatenate([lin_f_b[l][cp], lin_s_b[l][cp]]))
    wea = jnp.concatenate(wea_cols, axis=1)
    bea1 = jnp.concatenate(bea_cols)
    zb = jnp.zeros_like(wea)
    wea2 = jnp.concatenate(
        [jnp.concatenate([wea, zb], axis=1), jnp.concatenate([zb, wea], axis=1)],
        axis=0)
    bea2 = _pad8(jnp.concatenate([bea1, bea1]))
    ea6 = _ea_call(edge_attr.reshape(EE // 2, 32), wea2, bea2)

    h, st = _embed_call(x, emb_W, _pad8(emb_b))

    for l in range(NLAYER):
        w1f = lin_f_W[l][0:64]
        w2f = lin_f_W[l][64:128]
        w1s = lin_s_W[l][0:64]
        w2s = lin_s_W[l][64:128]
        cols = []
        for p in (0, 1):
            cp = slice(32 * p, 32 * p + 32)
            cols.append(jnp.concatenate([w1f[:, cp], w1s[:, cp]], axis=1))
            cols.append(jnp.concatenate([w2f[:, cp], w2s[:, cp]], axis=1))
        w256 = jnp.concatenate(cols, axis=1)
        bnp1 = jnp.zeros((8, FF), jnp.float32).at[0].set(bn1_g[l]).at[1].set(bn1_b[l])
        hbn, ps0, pd0, ps1, pd1 = _bnproj_call(h, st, bnp1, w256)

        eout = _edge_stage(src, dst, zeros_rpt, ps0, pd0, ea6[2 * l],
                           ps1, pd1, ea6[2 * l + 1])
        agg, st2 = _combine_call(eout, eout, eout, eout)
        bnp2 = jnp.zeros((8, FF), jnp.float32).at[0].set(bn2_g[l]).at[1].set(bn2_b[l])
        h, st = _post_call(agg, st2, bnp2, hbn)

    batch3d = batch.reshape(NRB, 1, RB)
    pooled, counts = _pool_call(h, batch3d)
    out = _head_call(pooled, counts, fc1_W, _pad8(fc1_b), out_W,
                     jnp.zeros((8, 8), jnp.float32).at[0, 0].set(out_b[0]))
    return out
